# baseline (speedup 1.0000x reference)
import jax
import jax.numpy as jnp
from jax import lax
from jax.experimental import pallas as pl
from jax.experimental.pallas import tpu as pltpu

S, D, E, CAP = 2048, 2048, 8, 512
BS_MM = 512
BS_OUT = 256


def _gate_body(hs_ref, wg_ref, p_ref, idx_ref, tot_ref, laux_ref,
               tril_ref, carry_ref):
    i = pl.program_id(0)

    @pl.when(i == 0)
    def _init():
        r = lax.broadcasted_iota(jnp.int32, (BS_MM, BS_MM), 0)
        c = lax.broadcasted_iota(jnp.int32, (BS_MM, BS_MM), 1)
        tril_ref[...] = (c <= r).astype(jnp.float32)
        carry_ref[...] = jnp.zeros((3, E), jnp.float32)

    x = jnp.dot(hs_ref[...], wg_ref[...],
                preferred_element_type=jnp.float32)
    eio = lax.broadcasted_iota(jnp.int32, (BS_MM, E), 1)
    m1 = jnp.max(x, axis=1, keepdims=True)
    e1 = jnp.min(jnp.where(x == m1, eio, E), axis=1, keepdims=True)
    mask1 = eio == e1
    xm = jnp.where(mask1, -jnp.inf, x)
    m2 = jnp.max(xm, axis=1, keepdims=True)
    e2 = jnp.min(jnp.where(xm == m2, eio, E), axis=1, keepdims=True)
    mask2 = eio == e2
    ex = jnp.exp(x - m1)
    z = jnp.sum(ex, axis=1, keepdims=True)
    gates = ex / z
    m1f = mask1.astype(jnp.float32)
    m2f = mask2.astype(jnp.float32)
    cs1 = jnp.dot(tril_ref[...], m1f, preferred_element_type=jnp.float32)
    cs2 = jnp.dot(tril_ref[...], m2f, preferred_element_type=jnp.float32)
    carry = carry_ref[...]
    c1row, c2row, gsrow = carry[0:1], carry[1:2], carry[2:3]
    loc1 = cs1 - 1.0 + c1row
    loc2 = cs2 - 1.0 + c2row
    r1 = jnp.sum(loc1 * m1f, axis=1, keepdims=True)
    r2 = jnp.sum(loc2 * m2f, axis=1, keepdims=True)
    p1 = jnp.sum(gates * m1f, axis=1, keepdims=True)
    p2 = jnp.sum(gates * m2f, axis=1, keepdims=True)
    new_c1 = c1row + cs1[BS_MM - 1:BS_MM, :]
    new_c2 = c2row + cs2[BS_MM - 1:BS_MM, :]
    new_gs = gsrow + jnp.sum(gates, axis=0, keepdims=True)
    carry_ref[...] = jnp.concatenate([new_c1, new_c2, new_gs], axis=0)
    p_ref[...] = jnp.concatenate([p1, p2, r1, r2], axis=1)
    idx_ref[...] = jnp.concatenate([e1, e2], axis=1)
    tot_ref[...] = new_c1
    laux_ref[...] = (jnp.sum(new_gs * new_c1) * (E / (S * S))).reshape(1, 1)


def _valcol(p_ref, idx_ref, tot_ref, n):
    p = p_ref[...]
    p1, p2 = p[:, 0:1], p[:, 1:2]
    r1, r2p = p[:, 2:3], p[:, 3:4]
    e1, e2 = idx_ref[...][:, 0:1], idx_ref[...][:, 1:2]
    eio = lax.broadcasted_iota(jnp.int32, (n, E), 1)
    tot1_at_e2 = jnp.sum(jnp.where(eio == e2, tot_ref[...], 0.0),
                         axis=1, keepdims=True)
    r2 = r2p + tot1_at_e2
    k1 = r1 < CAP
    k2 = r2 < CAP
    g1s = jnp.where(k1, p1, 0.0)
    g2s = jnp.where(k2, p2, 0.0)
    den = g1s + g2s
    den = jnp.where(den < 1e-9, 1e-9, den)
    g1 = g1s / den
    g2 = g2s / den
    c1 = jnp.where(k1, r1, 0.0).astype(jnp.int32)
    c2 = jnp.where(k2, r2, 0.0).astype(jnp.int32)
    is1 = eio == e1
    is2 = eio == e2
    val8 = jnp.where(is1, g1, 0.0) + jnp.where(is2, g2, 0.0)
    col8 = (jnp.where(is1, c1, 0) + jnp.where(is2, c2, 0)
            + jnp.where(is1 | is2, 0, CAP))
    return val8, col8


def _cw_body(p_ref, idx_ref, tot_ref, cw_ref):
    val8, col8 = _valcol(p_ref, idx_ref, tot_ref, BS_OUT)
    val = val8.reshape(BS_OUT, E, 1)
    col = col8.reshape(BS_OUT, E, 1)
    cio = lax.broadcasted_iota(jnp.int32, (BS_OUT, E, CAP), 2)
    hit = cio == col
    cw_ref[...] = jnp.where(
        hit, jnp.broadcast_to(val, (BS_OUT, E, CAP)), 0.0)


def _dm_body(p_ref, idx_ref, tot_ref, dm_ref):
    val8, col8 = _valcol(p_ref, idx_ref, tot_ref, BS_OUT)
    vpos = (val8 > 0.0).astype(jnp.int32).reshape(BS_OUT, E, 1)
    col = col8.reshape(BS_OUT, E, 1)
    cio = lax.broadcasted_iota(jnp.int32, (BS_OUT, E, CAP), 2)
    hit = cio == col
    dm_ref[...] = jnp.where(
        hit, jnp.broadcast_to(vpos, (BS_OUT, E, CAP)), 0).astype(jnp.int8)


def kernel(hidden_states, wg):
    pvals, idx, tot1, laux = pl.pallas_call(
        _gate_body,
        grid=(S // BS_MM,),
        in_specs=[
            pl.BlockSpec((BS_MM, D), lambda i: (i, 0)),
            pl.BlockSpec((D, E), lambda i: (0, 0)),
        ],
        out_specs=[
            pl.BlockSpec((BS_MM, 4), lambda i: (i, 0)),
            pl.BlockSpec((BS_MM, 2), lambda i: (i, 0)),
            pl.BlockSpec((1, E), lambda i: (0, 0)),
            pl.BlockSpec((1, 1), lambda i: (0, 0)),
        ],
        out_shape=[
            jax.ShapeDtypeStruct((S, 4), jnp.float32),
            jax.ShapeDtypeStruct((S, 2), jnp.int32),
            jax.ShapeDtypeStruct((1, E), jnp.float32),
            jax.ShapeDtypeStruct((1, 1), jnp.float32),
        ],
        scratch_shapes=[
            pltpu.VMEM((BS_MM, BS_MM), jnp.float32),
            pltpu.VMEM((3, E), jnp.float32),
        ],
    )(hidden_states, wg)

    in_specs_pt = [
        pl.BlockSpec((BS_OUT, 4), lambda i: (i, 0)),
        pl.BlockSpec((BS_OUT, 2), lambda i: (i, 0)),
        pl.BlockSpec((1, E), lambda i: (0, 0)),
    ]
    dm8 = pl.pallas_call(
        _dm_body,
        grid=(S // BS_OUT,),
        in_specs=in_specs_pt,
        out_specs=pl.BlockSpec((BS_OUT, E, CAP), lambda i: (i, 0, 0)),
        out_shape=jax.ShapeDtypeStruct((S, E, CAP), jnp.int8),
    )(pvals, idx, tot1)
    cw = pl.pallas_call(
        _cw_body,
        grid=(S // BS_OUT,),
        in_specs=in_specs_pt,
        out_specs=pl.BlockSpec((BS_OUT, E, CAP), lambda i: (i, 0, 0)),
        out_shape=jax.ShapeDtypeStruct((S, E, CAP), jnp.float32),
    )(pvals, idx, tot1)

    return (laux[0, 0], cw, dm8.view(jnp.bool_))


# speedup vs baseline: 1.1802x; 1.1802x over previous
import jax
import jax.numpy as jnp
from jax import lax
from jax.experimental import pallas as pl
from jax.experimental.pallas import tpu as pltpu

S, D, E, CAP = 2048, 2048, 8, 512
BS_MM = 512
BS_OUT = 512


def _gate_body(hs_ref, wg_ref, p_ref, idx_ref, tot_ref, laux_ref,
               tril_ref, carry_ref):
    i = pl.program_id(0)

    @pl.when(i == 0)
    def _init():
        r = lax.broadcasted_iota(jnp.int32, (BS_MM, BS_MM), 0)
        c = lax.broadcasted_iota(jnp.int32, (BS_MM, BS_MM), 1)
        tril_ref[...] = (c <= r).astype(jnp.float32)
        carry_ref[...] = jnp.zeros((3, E), jnp.float32)

    x = jnp.dot(hs_ref[...], wg_ref[...],
                preferred_element_type=jnp.float32)
    eio = lax.broadcasted_iota(jnp.int32, (BS_MM, E), 1)
    m1 = jnp.max(x, axis=1, keepdims=True)
    e1 = jnp.min(jnp.where(x == m1, eio, E), axis=1, keepdims=True)
    mask1 = eio == e1
    xm = jnp.where(mask1, -jnp.inf, x)
    m2 = jnp.max(xm, axis=1, keepdims=True)
    e2 = jnp.min(jnp.where(xm == m2, eio, E), axis=1, keepdims=True)
    mask2 = eio == e2
    ex = jnp.exp(x - m1)
    z = jnp.sum(ex, axis=1, keepdims=True)
    gates = ex / z
    m1f = mask1.astype(jnp.float32)
    m2f = mask2.astype(jnp.float32)
    cs1 = jnp.dot(tril_ref[...], m1f, preferred_element_type=jnp.float32)
    cs2 = jnp.dot(tril_ref[...], m2f, preferred_element_type=jnp.float32)
    carry = carry_ref[...]
    c1row, c2row, gsrow = carry[0:1], carry[1:2], carry[2:3]
    loc1 = cs1 - 1.0 + c1row
    loc2 = cs2 - 1.0 + c2row
    r1 = jnp.sum(loc1 * m1f, axis=1, keepdims=True)
    r2 = jnp.sum(loc2 * m2f, axis=1, keepdims=True)
    p1 = jnp.sum(gates * m1f, axis=1, keepdims=True)
    p2 = jnp.sum(gates * m2f, axis=1, keepdims=True)
    new_c1 = c1row + cs1[BS_MM - 1:BS_MM, :]
    new_c2 = c2row + cs2[BS_MM - 1:BS_MM, :]
    new_gs = gsrow + jnp.sum(gates, axis=0, keepdims=True)
    carry_ref[...] = jnp.concatenate([new_c1, new_c2, new_gs], axis=0)
    p_ref[...] = jnp.concatenate([p1, p2, r1, r2], axis=1)
    idx_ref[...] = jnp.concatenate([e1, e2], axis=1)
    tot_ref[...] = new_c1
    laux_ref[...] = (jnp.sum(new_gs * new_c1) * (E / (S * S))).reshape(1, 1)


def _valcol(p_ref, idx_ref, tot_ref, n):
    p = p_ref[...]
    p1, p2 = p[:, 0:1], p[:, 1:2]
    r1, r2p = p[:, 2:3], p[:, 3:4]
    e1, e2 = idx_ref[...][:, 0:1], idx_ref[...][:, 1:2]
    eio = lax.broadcasted_iota(jnp.int32, (n, E), 1)
    tot1_at_e2 = jnp.sum(jnp.where(eio == e2, tot_ref[...], 0.0),
                         axis=1, keepdims=True)
    r2 = r2p + tot1_at_e2
    k1 = r1 < CAP
    k2 = r2 < CAP
    g1s = jnp.where(k1, p1, 0.0)
    g2s = jnp.where(k2, p2, 0.0)
    den = g1s + g2s
    den = jnp.where(den < 1e-9, 1e-9, den)
    g1 = g1s / den
    g2 = g2s / den
    c1 = jnp.where(k1, r1, 0.0).astype(jnp.int32)
    c2 = jnp.where(k2, r2, 0.0).astype(jnp.int32)
    is1 = eio == e1
    is2 = eio == e2
    val8 = jnp.where(is1, g1, 0.0) + jnp.where(is2, g2, 0.0)
    col8 = (jnp.where(is1, c1, 0) + jnp.where(is2, c2, 0)
            + jnp.where(is1 | is2, 0, CAP))
    return val8, col8


def _dense_body(p_ref, idx_ref, tot_ref, cw_ref, dm_ref):
    val8, col8 = _valcol(p_ref, idx_ref, tot_ref, BS_OUT)
    val = val8.reshape(BS_OUT, E, 1)
    col = col8.reshape(BS_OUT, E, 1)
    cio = lax.broadcasted_iota(jnp.int32, (BS_OUT, E, CAP), 2)
    hit = cio == col
    v = jnp.where(hit, jnp.broadcast_to(val, (BS_OUT, E, CAP)), 0.0)
    cw_ref[...] = v
    dm_ref[...] = (v > 0.0).astype(jnp.int8)


def kernel(hidden_states, wg):
    pvals, idx, tot1, laux = pl.pallas_call(
        _gate_body,
        grid=(S // BS_MM,),
        in_specs=[
            pl.BlockSpec((BS_MM, D), lambda i: (i, 0)),
            pl.BlockSpec((D, E), lambda i: (0, 0)),
        ],
        out_specs=[
            pl.BlockSpec((BS_MM, 4), lambda i: (i, 0)),
            pl.BlockSpec((BS_MM, 2), lambda i: (i, 0)),
            pl.BlockSpec((1, E), lambda i: (0, 0)),
            pl.BlockSpec((1, 1), lambda i: (0, 0)),
        ],
        out_shape=[
            jax.ShapeDtypeStruct((S, 4), jnp.float32),
            jax.ShapeDtypeStruct((S, 2), jnp.int32),
            jax.ShapeDtypeStruct((1, E), jnp.float32),
            jax.ShapeDtypeStruct((1, 1), jnp.float32),
        ],
        scratch_shapes=[
            pltpu.VMEM((BS_MM, BS_MM), jnp.float32),
            pltpu.VMEM((3, E), jnp.float32),
        ],
    )(hidden_states, wg)

    cw, dm8 = pl.pallas_call(
        _dense_body,
        grid=(S // BS_OUT,),
        in_specs=[
            pl.BlockSpec((BS_OUT, 4), lambda i: (i, 0)),
            pl.BlockSpec((BS_OUT, 2), lambda i: (i, 0)),
            pl.BlockSpec((1, E), lambda i: (0, 0)),
        ],
        out_specs=[
            pl.BlockSpec((BS_OUT, E, CAP), lambda i: (i, 0, 0)),
            pl.BlockSpec((BS_OUT, E, CAP), lambda i: (i, 0, 0)),
        ],
        out_shape=[
            jax.ShapeDtypeStruct((S, E, CAP), jnp.float32),
            jax.ShapeDtypeStruct((S, E, CAP), jnp.int8),
        ],
    )(pvals, idx, tot1)

    return (laux[0, 0], cw, dm8.view(jnp.bool_))
